# Initial kernel scaffold; baseline (speedup 1.0000x reference)
import jax
import jax.numpy as jnp
from jax.experimental import pallas as pl


def kernel(anchors, deltas, scores, level_idxs):
    raise NotImplementedError("write your pallas kernel here")


# speedup vs baseline: 66.7224x; 66.7224x over previous
import jax
import jax.numpy as jnp
import numpy as np
from jax.experimental import pallas as pl
from jax.experimental.pallas import tpu as pltpu

N_TOP = 2000
N_PAD = 2048
IOU_THR = 0.7
IMG_H = 1024.0
IMG_W = 1024.0
MAX_RATIO = float(np.abs(np.log(16.0 / 1000.0)))
RB = 256


def _decode(x1, y1, x2, y2, dx, dy, dw, dh):
    px = (x1 + x2) * 0.5
    py = (y1 + y2) * 0.5
    pw = x2 - x1
    ph = y2 - y1
    dwc = jnp.clip(dw, -MAX_RATIO, MAX_RATIO)
    dhc = jnp.clip(dh, -MAX_RATIO, MAX_RATIO)
    gx = px + pw * dx
    gy = py + ph * dy
    gw = pw * jnp.exp(dwc)
    gh = ph * jnp.exp(dhc)
    nx1 = jnp.clip(gx - gw * 0.5, 0.0, IMG_W)
    ny1 = jnp.clip(gy - gh * 0.5, 0.0, IMG_H)
    nx2 = jnp.clip(gx + gw * 0.5, 0.0, IMG_W)
    ny2 = jnp.clip(gy + gh * 0.5, 0.0, IMG_H)
    return nx1, ny1, nx2, ny2


def _nms_kernel(at_ref, dt_ref, ac_ref, dc_ref, sc_ref, lvr_ref, lvc_ref,
                out_ref, s_ref, keep_ref):
    ar = at_ref[...]
    dr = dt_ref[...]
    x1r, y1r, x2r, y2r = _decode(ar[0:1], ar[1:2], ar[2:3], ar[3:4],
                                 dr[0:1], dr[1:2], dr[2:3], dr[3:4])
    ac = ac_ref[...]
    dc = dc_ref[...]
    x1c, y1c, x2c, y2c = _decode(ac[:, 0:1], ac[:, 1:2], ac[:, 2:3], ac[:, 3:4],
                                 dc[:, 0:1], dc[:, 1:2], dc[:, 2:3], dc[:, 3:4])

    mc = jnp.max(jnp.maximum(jnp.maximum(x1r, y1r), jnp.maximum(x2r, y2r)))
    off_r = lvr_ref[...] * (mc + 1.0)
    off_c = lvc_ref[...] * (mc + 1.0)

    bx1r = x1r + off_r
    by1r = y1r + off_r
    bx2r = x2r + off_r
    by2r = y2r + off_r
    bx1c = x1c + off_c
    by1c = y1c + off_c
    bx2c = x2c + off_c
    by2c = y2c + off_c
    area_r = (bx2r - bx1r) * (by2r - by1r)
    area_c = (bx2c - bx1c) * (by2c - by1c)

    for b in range(N_PAD // RB):
        rs = b * RB
        ltx = jnp.maximum(bx1c[rs:rs + RB, :], bx1r)
        lty = jnp.maximum(by1c[rs:rs + RB, :], by1r)
        rbx = jnp.minimum(bx2c[rs:rs + RB, :], bx2r)
        rby = jnp.minimum(by2c[rs:rs + RB, :], by2r)
        w = jnp.maximum(rbx - ltx, 0.0)
        h = jnp.maximum(rby - lty, 0.0)
        inter = w * h
        iou = inter / (area_c[rs:rs + RB, :] + area_r - inter + 1e-6)
        row_ids = jax.lax.broadcasted_iota(jnp.int32, (RB, N_PAD), 0) + rs
        col_ids = jax.lax.broadcasted_iota(jnp.int32, (RB, N_PAD), 1)
        s = (iou > IOU_THR) & (col_ids > row_ids)
        s_ref[rs:rs + RB, :] = s.astype(jnp.bfloat16)

    keep_ref[...] = jnp.ones((8, N_PAD), jnp.float32)

    def cond(c):
        return c[1]

    def body(c):
        it, _ = c
        k = keep_ref[...]
        v = jax.lax.dot_general(k.astype(jnp.bfloat16), s_ref[...],
                                (((1,), (0,)), ((), ())),
                                preferred_element_type=jnp.float32)
        k2 = (v == 0.0).astype(jnp.float32)
        changed = jnp.any(k2 != k)
        keep_ref[...] = k2
        return it + 1, changed

    jax.lax.while_loop(cond, body, (0, True))

    k1 = keep_ref[0:1, :]
    out_ref[...] = jnp.concatenate(
        [x1r * k1, y1r * k1, x2r * k1, y2r * k1, sc_ref[...] * k1,
         jnp.zeros((3, N_PAD), jnp.float32)], axis=0)


def kernel(anchors, deltas, scores, level_idxs):
    scores_top, inds = jax.lax.top_k(scores, N_TOP)
    a_top = jnp.take(anchors, inds, axis=0)
    d_top = jnp.take(deltas, inds, axis=0)
    l_top = jnp.take(level_idxs, inds, axis=0).astype(jnp.float32)
    pad = N_PAD - N_TOP
    ac = jnp.pad(a_top, ((0, pad), (0, 0)))
    dc = jnp.pad(d_top, ((0, pad), (0, 0)))
    sc = jnp.pad(scores_top, (0, pad))[None, :]
    lvr = jnp.pad(l_top, (0, pad))[None, :]
    out = pl.pallas_call(
        _nms_kernel,
        out_shape=jax.ShapeDtypeStruct((8, N_PAD), jnp.float32),
        scratch_shapes=[
            pltpu.VMEM((N_PAD, N_PAD), jnp.bfloat16),
            pltpu.VMEM((8, N_PAD), jnp.float32),
        ],
    )(ac.T, dc.T, ac, dc, sc, lvr, lvr.T)
    return out[:5, :N_TOP].T
